# baseline (speedup 1.0000x reference)
import functools

import jax
import jax.numpy as jnp
from jax.experimental import pallas as pl
from jax.experimental.pallas import tpu as pltpu


def _conv_matrix(w_sp, H, W, k):
    p = (k - 1) // 2
    di = jnp.arange(H)[:, None] - jnp.arange(H)[None, :]
    dj = jnp.arange(W)[:, None] - jnp.arange(W)[None, :]
    vi = jnp.abs(di) <= p
    vj = jnp.abs(dj) <= p
    ki = jnp.clip(di + p, 0, k - 1)
    kj = jnp.clip(dj + p, 0, k - 1)
    wk = w_sp[:, ki, :][:, :, :, kj]
    mask = vi[:, :, None, None] & vj[None, None, :, :]
    wk = jnp.where(mask[None], wk, 0.0)
    return wk.transpose(0, 1, 3, 2, 4).reshape(2 * H * W, H * W)


def _cbam_body(x_ref, w1_ref, w2_ref, km_ref, o_ref, *, inv_hw, inv_c):
    NB = x_ref.shape[0]
    f32 = jnp.float32

    xs = [x_ref[nb].astype(f32) for nb in range(NB)]
    cols = [jnp.sum(xc, axis=-1, keepdims=True) * inv_hw for xc in xs]
    cols += [jnp.max(xc, axis=-1, keepdims=True) for xc in xs]
    pooled = jnp.concatenate(cols, axis=1)
    h = jnp.dot(w1_ref[...], pooled, preferred_element_type=f32)
    h = jnp.maximum(h, 0.0)
    g = jnp.dot(w2_ref[...], h, preferred_element_type=f32)
    cgate = jax.nn.sigmoid(g[:, :NB] + g[:, NB:])

    x1s = []
    rows = []
    for nb in range(NB):
        x1 = xs[nb] * cgate[:, nb:nb + 1]
        x1s.append(x1)
        mean_map = jnp.sum(x1, axis=0, keepdims=True) * inv_c
        max_map = jnp.max(x1, axis=0, keepdims=True)
        rows.append(jnp.concatenate([mean_map, max_map], axis=1))
    m_all = jnp.concatenate(rows, axis=0)
    sgate = jax.nn.sigmoid(
        jnp.dot(m_all, km_ref[...], preferred_element_type=f32))

    out_dt = o_ref.dtype
    for nb in range(NB):
        o_ref[nb] = (x1s[nb] * sgate[nb:nb + 1, :]).astype(out_dt)


def kernel(x, w_fc1, w_fc2, w_sp):
    N, C, H, W = x.shape
    Cr = w_fc1.shape[0]
    k = w_sp.shape[-1]
    HW = H * W

    x_flat = x.reshape(N, C, HW)
    km = _conv_matrix(w_sp.astype(jnp.float32), H, W, k)

    NB = 4 if N % 4 == 0 else (2 if N % 2 == 0 else 1)

    body = functools.partial(_cbam_body, inv_hw=1.0 / HW, inv_c=1.0 / C)
    out_flat = pl.pallas_call(
        body,
        out_shape=jax.ShapeDtypeStruct((N, C, HW), x.dtype),
        grid=(N // NB,),
        in_specs=[
            pl.BlockSpec((NB, C, HW), lambda b: (b, 0, 0)),
            pl.BlockSpec((Cr, C), lambda b: (0, 0)),
            pl.BlockSpec((C, Cr), lambda b: (0, 0)),
            pl.BlockSpec((2 * HW, HW), lambda b: (0, 0)),
        ],
        out_specs=pl.BlockSpec((NB, C, HW), lambda b: (b, 0, 0)),
        compiler_params=pltpu.CompilerParams(
            dimension_semantics=("parallel",),
            vmem_limit_bytes=56 * 1024 * 1024),
    )(x_flat, w_fc1.astype(jnp.float32), w_fc2.astype(jnp.float32), km)
    return out_flat.reshape(N, C, H, W)


# speedup vs baseline: 1.4668x; 1.4668x over previous
import functools

import jax
import jax.numpy as jnp
from jax.experimental import pallas as pl
from jax.experimental.pallas import tpu as pltpu


def _conv_matrix(w_sp, H, W, k):
    p = (k - 1) // 2
    ks = jnp.arange(k)[:, None, None]
    A = (jnp.arange(H)[None, :, None] - jnp.arange(H)[None, None, :] + p
         == ks).astype(jnp.float32)
    B = (jnp.arange(W)[None, :, None] - jnp.arange(W)[None, None, :] + p
         == ks).astype(jnp.float32)
    K = jnp.einsum('ckl,kio,ljq->cijoq', w_sp, A, B)
    return K.reshape(2 * H * W, H * W)


def _cbam_body(x_ref, w1_ref, w2_ref, km_ref, o_ref, *, inv_hw, inv_c):
    NB = x_ref.shape[0]
    f32 = jnp.float32

    xs = [x_ref[nb].astype(f32) for nb in range(NB)]
    cols = [jnp.sum(xc, axis=-1, keepdims=True) * inv_hw for xc in xs]
    cols += [jnp.max(xc, axis=-1, keepdims=True) for xc in xs]
    pooled = jnp.concatenate(cols, axis=1)
    h = jnp.dot(w1_ref[...], pooled, preferred_element_type=f32)
    h = jnp.maximum(h, 0.0)
    g = jnp.dot(w2_ref[...], h, preferred_element_type=f32)
    cgate = jax.nn.sigmoid(g[:, :NB] + g[:, NB:])

    x1s = []
    rows = []
    for nb in range(NB):
        x1 = xs[nb] * cgate[:, nb:nb + 1]
        x1s.append(x1)
        mean_map = jnp.sum(x1, axis=0, keepdims=True) * inv_c
        max_map = jnp.max(x1, axis=0, keepdims=True)
        rows.append(jnp.concatenate([mean_map, max_map], axis=1))
    m_all = jnp.concatenate(rows, axis=0)
    sgate = jax.nn.sigmoid(
        jnp.dot(m_all, km_ref[...], preferred_element_type=f32))

    out_dt = o_ref.dtype
    for nb in range(NB):
        o_ref[nb] = (x1s[nb] * sgate[nb:nb + 1, :]).astype(out_dt)


def kernel(x, w_fc1, w_fc2, w_sp):
    N, C, H, W = x.shape
    Cr = w_fc1.shape[0]
    k = w_sp.shape[-1]
    HW = H * W

    x_flat = x.reshape(N, C, HW)
    km = _conv_matrix(w_sp.astype(jnp.float32), H, W, k)

    NB = 4 if N % 4 == 0 else (2 if N % 2 == 0 else 1)

    body = functools.partial(_cbam_body, inv_hw=1.0 / HW, inv_c=1.0 / C)
    out_flat = pl.pallas_call(
        body,
        out_shape=jax.ShapeDtypeStruct((N, C, HW), x.dtype),
        grid=(N // NB,),
        in_specs=[
            pl.BlockSpec((NB, C, HW), lambda b: (b, 0, 0)),
            pl.BlockSpec((Cr, C), lambda b: (0, 0)),
            pl.BlockSpec((C, Cr), lambda b: (0, 0)),
            pl.BlockSpec((2 * HW, HW), lambda b: (0, 0)),
        ],
        out_specs=pl.BlockSpec((NB, C, HW), lambda b: (b, 0, 0)),
        compiler_params=pltpu.CompilerParams(
            dimension_semantics=("parallel",),
            vmem_limit_bytes=56 * 1024 * 1024),
    )(x_flat, w_fc1.astype(jnp.float32), w_fc2.astype(jnp.float32), km)
    return out_flat.reshape(N, C, H, W)


# speedup vs baseline: 2.4166x; 1.6475x over previous
import functools

import jax
import jax.numpy as jnp
from jax.experimental import pallas as pl
from jax.experimental.pallas import tpu as pltpu


def _cbam_body(x_ref, w1_ref, w2_ref, wc_ref, o_ref, *,
               inv_hw, inv_c, height, width, ksize):
    NB = x_ref.shape[0]
    H, W, k = height, width, ksize
    HW = H * W
    p = (k - 1) // 2
    f32 = jnp.float32

    xs = [x_ref[nb].astype(f32) for nb in range(NB)]
    cols = [jnp.sum(xc, axis=-1, keepdims=True) * inv_hw for xc in xs]
    cols += [jnp.max(xc, axis=-1, keepdims=True) for xc in xs]
    pooled = jnp.concatenate(cols, axis=1)
    h = jnp.dot(w1_ref[...], pooled, preferred_element_type=f32)
    h = jnp.maximum(h, 0.0)
    g = jnp.dot(w2_ref[...], h, preferred_element_type=f32)
    cgate = jax.nn.sigmoid(g[:, :NB] + g[:, NB:])

    x1s = []
    mean_rows = []
    max_rows = []
    for nb in range(NB):
        x1 = xs[nb] * cgate[:, nb:nb + 1]
        x1s.append(x1)
        mean_rows.append(jnp.sum(x1, axis=0, keepdims=True) * inv_c)
        max_rows.append(jnp.max(x1, axis=0, keepdims=True))
    m2 = jnp.concatenate(mean_rows + max_rows, axis=0)

    lane = jax.lax.broadcasted_iota(jnp.int32, (1, HW), 1)
    orow = lane // W
    ocol = lane % W
    rowm = [((orow + (ki - p) >= 0) & (orow + (ki - p) < H)).astype(f32)
            for ki in range(k)]
    colm = [((ocol + (kj - p) >= 0) & (ocol + (kj - p) < W)).astype(f32)
            for kj in range(k)]
    accs = [jnp.zeros((2 * NB, HW), f32) for _ in range(2)]
    for ki in range(k):
        for kj in range(k):
            t = ki * k + kj
            s = (ki - p) * W + (kj - p)
            rolled = pltpu.roll(m2, (-s) % HW, axis=1)
            wv = wc_ref[:, t:t + 1]
            accs[t % 2] = accs[t % 2] + (rolled * wv) * (rowm[ki] * colm[kj])
    conv = accs[0] + accs[1]
    sgate = jax.nn.sigmoid(conv[:NB] + conv[NB:])

    out_dt = o_ref.dtype
    for nb in range(NB):
        o_ref[nb] = (x1s[nb] * sgate[nb:nb + 1, :]).astype(out_dt)


def kernel(x, w_fc1, w_fc2, w_sp):
    N, C, H, W = x.shape
    Cr = w_fc1.shape[0]
    k = w_sp.shape[-1]
    HW = H * W

    NB = 4 if N % 4 == 0 else (2 if N % 2 == 0 else 1)

    x_flat = x.reshape(N, C, HW)
    wflat = w_sp.astype(jnp.float32).reshape(2, k * k)
    wcomb = jnp.concatenate(
        [jnp.tile(wflat[0:1], (NB, 1)), jnp.tile(wflat[1:2], (NB, 1))], axis=0)

    body = functools.partial(_cbam_body, inv_hw=1.0 / HW, inv_c=1.0 / C,
                             height=H, width=W, ksize=k)
    out_flat = pl.pallas_call(
        body,
        out_shape=jax.ShapeDtypeStruct((N, C, HW), x.dtype),
        grid=(N // NB,),
        in_specs=[
            pl.BlockSpec((NB, C, HW), lambda b: (b, 0, 0)),
            pl.BlockSpec((Cr, C), lambda b: (0, 0)),
            pl.BlockSpec((C, Cr), lambda b: (0, 0)),
            pl.BlockSpec((2 * NB, k * k), lambda b: (0, 0)),
        ],
        out_specs=pl.BlockSpec((NB, C, HW), lambda b: (b, 0, 0)),
        compiler_params=pltpu.CompilerParams(
            dimension_semantics=("parallel",),
            vmem_limit_bytes=48 * 1024 * 1024),
    )(x_flat, w_fc1.astype(jnp.float32), w_fc2.astype(jnp.float32), wcomb)
    return out_flat.reshape(N, C, H, W)
